# baseline (speedup 1.0000x reference)
import jax
import jax.numpy as jnp
from jax import lax
from jax.experimental import pallas as pl
from jax.experimental.pallas import tpu as pltpu
from jax.experimental.pallas import tpu_sc as plsc

B = 4096
NN = 50
NC = 50
NP = NN + NC
D = 64
LANES = 16

NW = 32
ROWS_PER_W = B // NW
CB = 16
NCHUNK = ROWS_PER_W // CB
CPAD = 56
ROWPAD = NN + CPAD


def _sc_body(x_hbm, off_hbm, tok_hbm, table_hbm, out_hbm,
             x_v, off_v, tok_v, idx_v, asm_v, sem, semo):
    wid = lax.axis_index("s") * 2 + lax.axis_index("c")
    base_row = wid * ROWS_PER_W
    ivec = lax.iota(jnp.int32, LANES)

    pltpu.sync_copy(tok_hbm, tok_v)
    pltpu.sync_copy(off_hbm, off_v)

    for b in range(CB):
        idx_v[b, pl.ds(48, LANES)] = jnp.zeros((LANES,), jnp.int32)

    @pl.loop(0, NCHUNK)
    def _chunk(ci):
        row0 = base_row + ci * CB
        pltpu.sync_copy(x_hbm.at[pl.ds(row0, CB)], x_v)

        @pl.loop(0, CB)
        def _idxrow(b):
            ib = jnp.full((LANES,), 0, jnp.int32) + b
            for j0 in (0, 16, 32, 34):
                ic = ivec + j0
                codes = plsc.load_gather(x_v, [ib, ic + NN])
                offs = plsc.load_gather(off_v, [ic])
                plsc.store_scatter(idx_v, [ib, ic],
                                   codes.astype(jnp.int32) + offs)

        cps = []
        for b in range(CB):
            cps.append(pltpu.async_copy(
                table_hbm.at[idx_v.at[b, pl.ds(0, CPAD)]],
                asm_v.at[b, pl.ds(NN, CPAD)], sem))

        for p in range(NN):
            tokv = [tok_v[p, pl.ds(LANES * dd, LANES)]
                    for dd in range(D // LANES)]
            ip = jnp.full((LANES,), p, jnp.int32)

            @pl.loop(0, CB, unroll=4)
            def _ncb(b, tokv=tokv, ip=ip, p=p):
                ib = jnp.full((LANES,), 0, jnp.int32) + b
                sv = plsc.load_gather(x_v, [ib, ip])
                for dd in range(D // LANES):
                    asm_v[b, p, pl.ds(LANES * dd, LANES)] = tokv[dd] * sv

        for cp in cps:
            cp.wait()

        @pl.loop(0, CB)
        def _outr(i):
            pltpu.async_copy(asm_v.at[i, pl.ds(0, NP)],
                             out_hbm.at[row0 + i], semo).wait()


@jax.jit
def _tokenize(x, off, tok, table):
    mesh = plsc.VectorSubcoreMesh(core_axis_name="c", subcore_axis_name="s",
                                  num_cores=2, num_subcores=16)
    f = pl.kernel(
        _sc_body,
        out_type=jax.ShapeDtypeStruct((B, NP, D), jnp.float32),
        mesh=mesh,
        scratch_types=[
            pltpu.VMEM((CB, NP), jnp.float32),
            pltpu.VMEM((NC,), jnp.int32),
            pltpu.VMEM((NN, D), jnp.float32),
            pltpu.VMEM((CB, 64), jnp.int32),
            pltpu.VMEM((CB, ROWPAD, D), jnp.float32),
            pltpu.SemaphoreType.DMA,
            pltpu.SemaphoreType.DMA,
        ],
        compiler_params=pltpu.CompilerParams(use_tc_tiling_on_sc=False,
                                             needs_layout_passes=False),
    )
    return f(x, off, tok, table)


def kernel(x, noncat_tokenizer, cat_table, noncat_idx, cat_idx, cat_offsets):
    return _tokenize(x, cat_offsets.astype(jnp.int32), noncat_tokenizer,
                     cat_table)


# speedup vs baseline: 2.2008x; 2.2008x over previous
import jax
import jax.numpy as jnp
from jax import lax
from jax.experimental import pallas as pl
from jax.experimental.pallas import tpu as pltpu
from jax.experimental.pallas import tpu_sc as plsc

B = 4096
NN = 50
NC = 50
NP = NN + NC
D = 64
LANES = 16

NW = 32
ROWS_PER_W = B // NW
CB = 16
NCHUNK = ROWS_PER_W // CB
NWIN = CB * NC // LANES


def _sc_body(x_hbm, off_hbm, tok_hbm, table_hbm, out_hbm,
             x_v, off_v, tok_v, cat_v, nc_v, sem, semo):
    wid = lax.axis_index("s") * 2 + lax.axis_index("c")
    base_row = wid * ROWS_PER_W
    ivec = lax.iota(jnp.int32, LANES)

    pltpu.sync_copy(tok_hbm, tok_v)
    pltpu.sync_copy(off_hbm, off_v)

    @pl.loop(0, NCHUNK)
    def _chunk(ci):
        row0 = base_row + ci * CB
        pltpu.sync_copy(x_hbm.at[pl.ds(row0, CB)], x_v)

        cps = []
        for t in range(NWIN):
            c = ivec + LANES * t
            q = c // NC
            r = c - q * NC
            codes = plsc.load_gather(x_v, [q, r + NN])
            offs = plsc.load_gather(off_v, [r])
            iv = codes.astype(jnp.int32) + offs
            cps.append(pltpu.async_copy(
                table_hbm.at[iv], cat_v.at[pl.ds(LANES * t, LANES)], sem))

        for p in range(NN):
            tokv = [tok_v[p, pl.ds(LANES * dd, LANES)]
                    for dd in range(D // LANES)]
            ip = jnp.full((LANES,), p, jnp.int32)

            @pl.loop(0, CB, unroll=4)
            def _ncb(b, tokv=tokv, ip=ip, p=p):
                ib = jnp.full((LANES,), 0, jnp.int32) + b
                sv = plsc.load_gather(x_v, [ib, ip])
                for dd in range(D // LANES):
                    nc_v[b * NN + p, pl.ds(LANES * dd, LANES)] = \
                        tokv[dd] * sv

        ocs = []
        for i in range(CB):
            ocs.append(pltpu.async_copy(
                nc_v.at[pl.ds(i * NN, NN)],
                out_hbm.at[row0 + i, pl.ds(0, NN)], semo))

        for cp in cps:
            cp.wait()

        for i in range(CB):
            ocs.append(pltpu.async_copy(
                cat_v.at[pl.ds(i * NC, NC)],
                out_hbm.at[row0 + i, pl.ds(NN, NC)], semo))

        for oc in ocs:
            oc.wait()


@jax.jit
def _tokenize(x, off, tok, table):
    mesh = plsc.VectorSubcoreMesh(core_axis_name="c", subcore_axis_name="s",
                                  num_cores=2, num_subcores=16)
    f = pl.kernel(
        _sc_body,
        out_type=jax.ShapeDtypeStruct((B, NP, D), jnp.float32),
        mesh=mesh,
        scratch_types=[
            pltpu.VMEM((CB, NP), jnp.float32),
            pltpu.VMEM((NC,), jnp.int32),
            pltpu.VMEM((NN, D), jnp.float32),
            pltpu.VMEM((CB * NC, D), jnp.float32),
            pltpu.VMEM((CB * NN, D), jnp.float32),
            pltpu.SemaphoreType.DMA,
            pltpu.SemaphoreType.DMA,
        ],
        compiler_params=pltpu.CompilerParams(use_tc_tiling_on_sc=False,
                                             needs_layout_passes=False),
    )
    return f(x, off, tok, table)


def kernel(x, noncat_tokenizer, cat_table, noncat_idx, cat_idx, cat_offsets):
    return _tokenize(x, cat_offsets.astype(jnp.int32), noncat_tokenizer,
                     cat_table)
